# baseline (speedup 1.0000x reference)
import functools

import jax
import jax.numpy as jnp
from jax import lax
from jax.experimental import pallas as pl
from jax.experimental.pallas import tpu as pltpu
from jax.experimental.pallas import tpu_sc as plsc

_M = 32
_NCOLS = 33
_LANES = 16
_NUM_CORES = 2
_NUM_SUBCORES = 16
_NUM_WORKERS = _NUM_CORES * _NUM_SUBCORES


def _make_sc_call(batch):
    rows_per_worker = batch // _NUM_WORKERS
    groups = rows_per_worker // _LANES
    mesh = plsc.VectorSubcoreMesh(
        core_axis_name="c", subcore_axis_name="s",
        num_cores=_NUM_CORES, num_subcores=_NUM_SUBCORES)

    @functools.partial(
        pl.kernel,
        mesh=mesh,
        out_type=jax.ShapeDtypeStruct((batch,), jnp.int32),
        scratch_types=[
            pltpu.VMEM((_M, rows_per_worker), jnp.int32),
            pltpu.VMEM((_NCOLS, _NCOLS), jnp.int32),
            pltpu.VMEM((rows_per_worker,), jnp.int32),
        ],
        compiler_params=pltpu.CompilerParams(
            needs_layout_passes=False,
            disable_bounds_checks=True,
            disable_semaphore_checks=True,
        ),
    )
    def sc_rank(xt_hbm, comb_hbm, out_hbm, x_v, comb_v, out_v):
        wid = (lax.axis_index("s") * jnp.int32(_NUM_CORES)
               + lax.axis_index("c"))
        rbase = wid * jnp.int32(rows_per_worker)
        pltpu.sync_copy(comb_hbm, comb_v)
        pltpu.sync_copy(xt_hbm.at[:, pl.ds(rbase, rows_per_worker)], x_v)

        def group_body(g, carry):
            gbase = g * jnp.int32(2 * _LANES)
            presum_a = jnp.zeros((_LANES,), jnp.int32)
            presum_b = jnp.zeros((_LANES,), jnp.int32)
            acc_a = jnp.zeros((_LANES,), jnp.int32)
            acc_b = jnp.zeros((_LANES,), jnp.int32)
            def step(i, c):
                presum_a, presum_b, acc_a, acc_b = c
                xi_a = x_v[i, pl.ds(gbase, _LANES)]
                xi_b = x_v[i, pl.ds(gbase + jnp.int32(_LANES), _LANES)]
                row_i = jnp.full((_LANES,), _M - 1, jnp.int32) - i
                cval_a = plsc.load_gather(comb_v, [row_i, presum_a])
                cval_b = plsc.load_gather(comb_v, [row_i, presum_b])
                acc_a = acc_a + cval_a * xi_a
                acc_b = acc_b + cval_b * xi_b
                presum_a = presum_a + xi_a
                presum_b = presum_b + xi_b
                return presum_a, presum_b, acc_a, acc_b

            presum_a, presum_b, acc_a, acc_b = lax.fori_loop(
                jnp.int32(0), jnp.int32(_M), step,
                (presum_a, presum_b, acc_a, acc_b))
            out_v[pl.ds(gbase, _LANES)] = acc_a
            out_v[pl.ds(gbase + jnp.int32(_LANES), _LANES)] = acc_b
            return carry

        lax.fori_loop(jnp.int32(0), jnp.int32(groups // 2), group_body,
                      jnp.int32(0))
        pltpu.sync_copy(out_v, out_hbm.at[pl.ds(rbase, rows_per_worker)])

    return sc_rank


@jax.jit
def kernel(x, comb):
    batch = x.shape[0]
    xt32 = x.T.astype(jnp.int32)
    comb32r = comb.astype(jnp.int32)[:, ::-1]
    out32 = _make_sc_call(batch)(xt32, comb32r)
    return out32.astype(jnp.int64)


# speedup vs baseline: 1.0532x; 1.0532x over previous
import functools

import jax
import jax.numpy as jnp
from jax import lax
from jax.experimental import pallas as pl
from jax.experimental.pallas import tpu as pltpu
from jax.experimental.pallas import tpu_sc as plsc

_M = 32
_NCOLS = 33
_LANES = 16
_NUM_CORES = 2
_NUM_SUBCORES = 16
_NUM_WORKERS = _NUM_CORES * _NUM_SUBCORES


def _make_sc_call(batch):
    rows_per_worker = batch // _NUM_WORKERS
    groups = rows_per_worker // _LANES
    mesh = plsc.VectorSubcoreMesh(
        core_axis_name="c", subcore_axis_name="s",
        num_cores=_NUM_CORES, num_subcores=_NUM_SUBCORES)

    @functools.partial(
        pl.kernel,
        mesh=mesh,
        out_type=jax.ShapeDtypeStruct((batch,), jnp.int32),
        scratch_types=[
            pltpu.VMEM((_M, rows_per_worker), jnp.int32),
            pltpu.VMEM((_NCOLS, _NCOLS), jnp.int32),
            pltpu.VMEM((rows_per_worker,), jnp.int32),
            pltpu.SemaphoreType.DMA,
            pltpu.SemaphoreType.DMA,
        ],
        compiler_params=pltpu.CompilerParams(
            needs_layout_passes=False,
            disable_bounds_checks=True,
            disable_semaphore_checks=True,
            skip_device_barrier=True,
        ),
    )
    def sc_rank(xt_hbm, comb_hbm, out_hbm, x_v, comb_v, out_v,
                sem_c, sem_x):
        wid = (lax.axis_index("s") * jnp.int32(_NUM_CORES)
               + lax.axis_index("c"))
        rbase = wid * jnp.int32(rows_per_worker)
        cpy_c = pltpu.async_copy(comb_hbm, comb_v, sem_c)
        cpy_x = pltpu.async_copy(
            xt_hbm.at[:, pl.ds(rbase, rows_per_worker)], x_v, sem_x)
        cpy_c.wait()
        cpy_x.wait()

        def group_body(g, carry):
            gbase = g * jnp.int32(2 * _LANES)
            presum_a = jnp.zeros((_LANES,), jnp.int32)
            presum_b = jnp.zeros((_LANES,), jnp.int32)
            acc_a = jnp.zeros((_LANES,), jnp.int32)
            acc_b = jnp.zeros((_LANES,), jnp.int32)
            for i in range(_M):
                xi_a = x_v[i, pl.ds(gbase, _LANES)]
                xi_b = x_v[i, pl.ds(gbase + jnp.int32(_LANES), _LANES)]
                row_i = jnp.full((_LANES,), _M - 1 - i, jnp.int32)
                left_a = jnp.full((_LANES,), _M, jnp.int32) - presum_a
                left_b = jnp.full((_LANES,), _M, jnp.int32) - presum_b
                cval_a = plsc.load_gather(comb_v, [row_i, left_a])
                cval_b = plsc.load_gather(comb_v, [row_i, left_b])
                acc_a = acc_a + cval_a * xi_a
                acc_b = acc_b + cval_b * xi_b
                presum_a = presum_a + xi_a
                presum_b = presum_b + xi_b
            out_v[pl.ds(gbase, _LANES)] = acc_a
            out_v[pl.ds(gbase + jnp.int32(_LANES), _LANES)] = acc_b
            return carry

        lax.fori_loop(jnp.int32(0), jnp.int32(groups // 2), group_body,
                      jnp.int32(0))
        pltpu.sync_copy(out_v, out_hbm.at[pl.ds(rbase, rows_per_worker)])

    return sc_rank


@jax.jit
def kernel(x, comb):
    batch = x.shape[0]
    xt32 = x.T.astype(jnp.int32)
    comb32 = comb.astype(jnp.int32)
    out32 = _make_sc_call(batch)(xt32, comb32)
    return out32.astype(jnp.int64)


# speedup vs baseline: 1.0877x; 1.0328x over previous
import functools

import jax
import jax.numpy as jnp
from jax import lax
from jax.experimental import pallas as pl
from jax.experimental.pallas import tpu as pltpu
from jax.experimental.pallas import tpu_sc as plsc

_M = 32
_NCOLS = 33
_LANES = 16
_NUM_CORES = 1
_NUM_SUBCORES = 16
_NUM_WORKERS = _NUM_CORES * _NUM_SUBCORES


def _make_sc_call(batch):
    rows_per_worker = batch // _NUM_WORKERS
    groups = rows_per_worker // _LANES
    mesh = plsc.VectorSubcoreMesh(
        core_axis_name="c", subcore_axis_name="s",
        num_cores=_NUM_CORES, num_subcores=_NUM_SUBCORES)

    @functools.partial(
        pl.kernel,
        mesh=mesh,
        out_type=jax.ShapeDtypeStruct((batch,), jnp.int32),
        scratch_types=[
            pltpu.VMEM((_M, rows_per_worker), jnp.int32),
            pltpu.VMEM((_NCOLS, _NCOLS), jnp.int32),
            pltpu.VMEM((rows_per_worker,), jnp.int32),
            pltpu.SemaphoreType.DMA,
            pltpu.SemaphoreType.DMA,
        ],
        compiler_params=pltpu.CompilerParams(
            needs_layout_passes=False,
            disable_bounds_checks=True,
            disable_semaphore_checks=True,
            skip_device_barrier=True,
        ),
    )
    def sc_rank(xt_hbm, comb_hbm, out_hbm, x_v, comb_v, out_v,
                sem_c, sem_x):
        wid = (lax.axis_index("s") * jnp.int32(_NUM_CORES)
               + lax.axis_index("c"))
        rbase = wid * jnp.int32(rows_per_worker)
        cpy_c = pltpu.async_copy(comb_hbm, comb_v, sem_c)
        cpy_x = pltpu.async_copy(
            xt_hbm.at[:, pl.ds(rbase, rows_per_worker)], x_v, sem_x)
        cpy_c.wait()
        cpy_x.wait()

        def group_body(g, carry):
            gbase = g * jnp.int32(2 * _LANES)
            presum_a = jnp.zeros((_LANES,), jnp.int32)
            presum_b = jnp.zeros((_LANES,), jnp.int32)
            acc_a = jnp.zeros((_LANES,), jnp.int32)
            acc_b = jnp.zeros((_LANES,), jnp.int32)
            for i in range(_M):
                xi_a = x_v[i, pl.ds(gbase, _LANES)]
                xi_b = x_v[i, pl.ds(gbase + jnp.int32(_LANES), _LANES)]
                row_i = jnp.full((_LANES,), _M - 1 - i, jnp.int32)
                left_a = jnp.full((_LANES,), _M, jnp.int32) - presum_a
                left_b = jnp.full((_LANES,), _M, jnp.int32) - presum_b
                cval_a = plsc.load_gather(comb_v, [row_i, left_a])
                cval_b = plsc.load_gather(comb_v, [row_i, left_b])
                acc_a = acc_a + cval_a * xi_a
                acc_b = acc_b + cval_b * xi_b
                presum_a = presum_a + xi_a
                presum_b = presum_b + xi_b
            out_v[pl.ds(gbase, _LANES)] = acc_a
            out_v[pl.ds(gbase + jnp.int32(_LANES), _LANES)] = acc_b
            return carry

        lax.fori_loop(jnp.int32(0), jnp.int32(groups // 2), group_body,
                      jnp.int32(0))
        pltpu.sync_copy(out_v, out_hbm.at[pl.ds(rbase, rows_per_worker)])

    return sc_rank


@jax.jit
def kernel(x, comb):
    batch = x.shape[0]
    xt32 = x.T.astype(jnp.int32)
    comb32 = comb.astype(jnp.int32)
    out32 = _make_sc_call(batch)(xt32, comb32)
    return out32.astype(jnp.int64)
